# Initial kernel scaffold; baseline (speedup 1.0000x reference)
import functools

import jax
import jax.numpy as jnp
from jax import lax
from jax.experimental import pallas as pl
from jax.experimental.pallas import tpu as pltpu
from jax.experimental.pallas import tpu_sc as plsc

N = 10000
F = 128
E = 320000
NP = 10240
LANES = 16
NCORES = 2
NSUB = 16
NWORK = NCORES * NSUB
CHUNK = 128
CPW = 79
EPW = CPW * CHUNK
EPAD = NWORK * EPW
ROWS_PT = NP // NSUB

_mesh = plsc.VectorSubcoreMesh(core_axis_name="c", subcore_axis_name="s")


def _worker(c, s):
    return c * NSUB + s


@functools.partial(
    pl.kernel,
    out_type=jax.ShapeDtypeStruct((NCORES, NP), jnp.float32),
    mesh=_mesh,
    scratch_types=[
        pltpu.VMEM((CPW, CHUNK), jnp.int32),
        pltpu.VMEM((CPW, CHUNK), jnp.float32),
        pltpu.VMEM((ROWS_PT,), jnp.float32),
        pltpu.VMEM_SHARED((NP,), jnp.float32),
    ],
)
def _deg_kernel(dstr_hbm, maskr_hbm, out_hbm, dst_v, mask_v, zero_v, deg_sh):
    c = lax.axis_index("c")
    s = lax.axis_index("s")
    w = _worker(c, s)

    @pl.loop(0, ROWS_PT // LANES)
    def _(i):
        zero_v[pl.ds(i * LANES, LANES)] = jnp.zeros((LANES,), jnp.float32)

    pltpu.sync_copy(zero_v, deg_sh.at[pl.ds(s * ROWS_PT, ROWS_PT)])
    plsc.subcore_barrier()

    base = w * CPW
    pltpu.sync_copy(dstr_hbm.at[pl.ds(base, CPW), :], dst_v)
    pltpu.sync_copy(maskr_hbm.at[pl.ds(base, CPW), :], mask_v)

    @pl.loop(0, CPW)
    def _(j):
        pltpu.sync_copy(mask_v.at[j], deg_sh.at[dst_v.at[j]], add=True)

    plsc.subcore_barrier()
    pltpu.sync_copy(deg_sh.at[pl.ds(s * ROWS_PT, ROWS_PT)],
                    out_hbm.at[c, pl.ds(s * ROWS_PT, ROWS_PT)])


@functools.partial(
    pl.kernel,
    out_type=jax.ShapeDtypeStruct((NCORES, NP, F), jnp.float32),
    mesh=_mesh,
    scratch_types=[
        pltpu.VMEM((CPW, CHUNK), jnp.int32),
        pltpu.VMEM((CPW, CHUNK), jnp.int32),
        pltpu.VMEM((CPW, CHUNK), jnp.float32),
        pltpu.VMEM((CHUNK, F), jnp.float32),
        pltpu.VMEM_SHARED((NP, F), jnp.float32),
    ],
)
def _agg_kernel(hh_hbm, srcr_hbm, dstr_hbm, maskr_hbm, out_hbm,
                src_v, dst_v, mask_v, rows_v, agg_sh):
    c = lax.axis_index("c")
    s = lax.axis_index("s")
    w = _worker(c, s)

    @pl.loop(0, CHUNK)
    def _(i):
        for v in range(F // LANES):
            rows_v[i, pl.ds(v * LANES, LANES)] = jnp.zeros((LANES,), jnp.float32)

    for k in range(ROWS_PT // CHUNK):
        pltpu.sync_copy(rows_v,
                        agg_sh.at[pl.ds(s * ROWS_PT + k * CHUNK, CHUNK), :])
    plsc.subcore_barrier()

    base = w * CPW
    pltpu.sync_copy(srcr_hbm.at[pl.ds(base, CPW), :], src_v)
    pltpu.sync_copy(dstr_hbm.at[pl.ds(base, CPW), :], dst_v)
    pltpu.sync_copy(maskr_hbm.at[pl.ds(base, CPW), :], mask_v)

    @pl.loop(0, CPW)
    def _(j):
        pltpu.sync_copy(hh_hbm.at[src_v.at[j]], rows_v)

        @pl.loop(0, CHUNK)
        def _(e):
            m = mask_v[j, e]
            for v in range(F // LANES):
                sl = pl.ds(v * LANES, LANES)
                rows_v[e, sl] = rows_v[e, sl] * m

        pltpu.sync_copy(rows_v, agg_sh.at[dst_v.at[j]], add=True)

    plsc.subcore_barrier()
    pltpu.sync_copy(agg_sh.at[pl.ds(s * ROWS_PT, ROWS_PT), :],
                    out_hbm.at[c, pl.ds(s * ROWS_PT, ROWS_PT), :])


_BLK = 1024
_GRID = NP // _BLK


def _dinv_block(degp):
    deg = degp[0, :] + degp[1, :]
    return jax.lax.rsqrt(jnp.maximum(deg, 1.0))[:, None]


def _dense_a_body(degp_ref, feat_ref, hh_ref):
    hh_ref[...] = feat_ref[...] * _dinv_block(degp_ref[...])


def _dense_a(degp, feat_p):
    return pl.pallas_call(
        _dense_a_body,
        grid=(_GRID,),
        in_specs=[
            pl.BlockSpec((NCORES, _BLK), lambda i: (0, i)),
            pl.BlockSpec((_BLK, F), lambda i: (i, 0)),
        ],
        out_specs=pl.BlockSpec((_BLK, F), lambda i: (i, 0)),
        out_shape=jax.ShapeDtypeStruct((NP, F), jnp.float32),
    )(degp, feat_p)


def _dense_b_body(degp_ref, feat_ref, aggp_ref, feat2_ref, h2_ref, hh2_ref):
    dinv = _dinv_block(degp_ref[...])
    feat = feat_ref[...]
    agg = aggp_ref[0] + aggp_ref[1]
    feat2 = feat - agg * dinv
    feat2_ref[...] = feat2
    h2_ref[...] = 0.5 * feat + 0.4 * feat2
    hh2_ref[...] = feat2 * dinv


def _dense_b(degp, feat_p, aggp):
    return pl.pallas_call(
        _dense_b_body,
        grid=(_GRID,),
        in_specs=[
            pl.BlockSpec((NCORES, _BLK), lambda i: (0, i)),
            pl.BlockSpec((_BLK, F), lambda i: (i, 0)),
            pl.BlockSpec((NCORES, _BLK, F), lambda i: (0, i, 0)),
        ],
        out_specs=[pl.BlockSpec((_BLK, F), lambda i: (i, 0))] * 3,
        out_shape=[jax.ShapeDtypeStruct((NP, F), jnp.float32)] * 3,
    )(degp, feat_p, aggp)


def _dense_c_body(degp_ref, feat2_ref, h2_ref, aggp_ref, h_ref):
    dinv = _dinv_block(degp_ref[...])
    agg = aggp_ref[0] + aggp_ref[1]
    feat3 = feat2_ref[...] - agg * dinv
    h_ref[...] = h2_ref[...] + 0.1 * feat3


def _dense_c(degp, feat2, h2, aggp):
    return pl.pallas_call(
        _dense_c_body,
        grid=(_GRID,),
        in_specs=[
            pl.BlockSpec((NCORES, _BLK), lambda i: (0, i)),
            pl.BlockSpec((_BLK, F), lambda i: (i, 0)),
            pl.BlockSpec((_BLK, F), lambda i: (i, 0)),
            pl.BlockSpec((NCORES, _BLK, F), lambda i: (0, i, 0)),
        ],
        out_specs=pl.BlockSpec((_BLK, F), lambda i: (i, 0)),
        out_shape=jax.ShapeDtypeStruct((NP, F), jnp.float32),
    )(degp, feat2, h2, aggp)


def kernel(feat, edge_index, edge_mask):
    src = edge_index[0].astype(jnp.int32)
    dst = edge_index[1].astype(jnp.int32)
    pad = EPAD - E
    srcr = jnp.pad(src, (0, pad)).reshape(NWORK * CPW, CHUNK)
    dstr = jnp.pad(dst, (0, pad)).reshape(NWORK * CPW, CHUNK)
    maskr = jnp.pad(edge_mask, (0, pad)).reshape(NWORK * CPW, CHUNK)
    feat_p = jnp.pad(feat, ((0, NP - N), (0, 0)))

    degp = _deg_kernel(dstr, maskr)
    hh = _dense_a(degp, feat_p)
    aggp1 = _agg_kernel(hh, srcr, dstr, maskr)
    feat2, h2, hh2 = _dense_b(degp, feat_p, aggp1)
    aggp2 = _agg_kernel(hh2, srcr, dstr, maskr)
    h = _dense_c(degp, feat2, h2, aggp2)
    return h[:N]


# speedup vs baseline: 3.2837x; 3.2837x over previous
import functools

import jax
import jax.numpy as jnp
from jax import lax
from jax.experimental import pallas as pl
from jax.experimental.pallas import tpu as pltpu
from jax.experimental.pallas import tpu_sc as plsc

N = 10000
F = 128
E = 320000
NP = 10240
LANES = 16
NCORES = 2
NSUB = 16
NWORK = NCORES * NSUB
CHUNK = 128
CPW = 80
EPW = CPW * CHUNK
EPAD = NWORK * EPW
ROWS_PT = NP // NSUB

_mesh = plsc.VectorSubcoreMesh(core_axis_name="c", subcore_axis_name="s")


def _worker(c, s):
    return c * NSUB + s


@functools.partial(
    pl.kernel,
    out_type=jax.ShapeDtypeStruct((NCORES * NP,), jnp.float32),
    mesh=_mesh,
    scratch_types=[
        pltpu.VMEM((CPW, CHUNK), jnp.int32),
        pltpu.VMEM((CPW, CHUNK), jnp.float32),
        pltpu.VMEM((ROWS_PT,), jnp.float32),
        pltpu.VMEM_SHARED((NP,), jnp.float32),
    ],
)
def _deg_kernel(dstr_hbm, maskr_hbm, out_hbm, dst_v, mask_v, zero_v, deg_sh):
    c = lax.axis_index("c")
    s = lax.axis_index("s")
    w = _worker(c, s)

    @pl.loop(0, ROWS_PT // LANES)
    def _(i):
        zero_v[pl.ds(i * LANES, LANES)] = jnp.zeros((LANES,), jnp.float32)

    pltpu.sync_copy(zero_v, deg_sh.at[pl.ds(s * ROWS_PT, ROWS_PT)])
    plsc.subcore_barrier()

    base = w * CPW
    pltpu.sync_copy(dstr_hbm.at[pl.ds(base, CPW), :], dst_v)
    pltpu.sync_copy(maskr_hbm.at[pl.ds(base, CPW), :], mask_v)

    @pl.loop(0, CPW)
    def _(j):
        pltpu.sync_copy(mask_v.at[j], deg_sh.at[dst_v.at[j]], add=True)

    plsc.subcore_barrier()
    pltpu.sync_copy(deg_sh.at[pl.ds(s * ROWS_PT, ROWS_PT)],
                    out_hbm.at[pl.ds(c * NP + s * ROWS_PT, ROWS_PT)])


@functools.partial(
    pl.kernel,
    out_type=jax.ShapeDtypeStruct((NCORES, NP, F), jnp.float32),
    mesh=_mesh,
    scratch_types=[
        pltpu.VMEM((CPW, CHUNK), jnp.int32),
        pltpu.VMEM((CPW, CHUNK), jnp.int32),
        pltpu.VMEM((CPW, CHUNK), jnp.float32),
        pltpu.VMEM((CHUNK, F), jnp.float32),
        pltpu.VMEM_SHARED((NP, F), jnp.float32),
    ],
)
def _agg_kernel(hh_hbm, srcr_hbm, dstr_hbm, maskr_hbm, out_hbm,
                src_v, dst_v, mask_v, rows_v, agg_sh):
    c = lax.axis_index("c")
    s = lax.axis_index("s")
    w = _worker(c, s)

    @pl.loop(0, CHUNK)
    def _(i):
        for v in range(F // LANES):
            rows_v[i, pl.ds(v * LANES, LANES)] = jnp.zeros((LANES,), jnp.float32)

    for k in range(ROWS_PT // CHUNK):
        pltpu.sync_copy(rows_v,
                        agg_sh.at[pl.ds(s * ROWS_PT + k * CHUNK, CHUNK), :])
    plsc.subcore_barrier()

    base = w * CPW
    pltpu.sync_copy(srcr_hbm.at[pl.ds(base, CPW), :], src_v)
    pltpu.sync_copy(dstr_hbm.at[pl.ds(base, CPW), :], dst_v)
    pltpu.sync_copy(maskr_hbm.at[pl.ds(base, CPW), :], mask_v)

    @pl.loop(0, CPW)
    def _(j):
        pltpu.sync_copy(hh_hbm.at[src_v.at[j]], rows_v)

        @pl.loop(0, CHUNK // LANES)
        def _(g):
            mv = mask_v[j, pl.ds(g * LANES, LANES)]
            for l in range(LANES):
                m = mv[l]
                e = g * LANES + l
                for v in range(F // LANES):
                    sl = pl.ds(v * LANES, LANES)
                    rows_v[e, sl] = rows_v[e, sl] * m

        pltpu.sync_copy(rows_v, agg_sh.at[dst_v.at[j]], add=True)

    plsc.subcore_barrier()
    pltpu.sync_copy(agg_sh.at[pl.ds(s * ROWS_PT, ROWS_PT), :],
                    out_hbm.at[c, pl.ds(s * ROWS_PT, ROWS_PT), :])


_BLK = 1024
_GRID = NP // _BLK


def _dinv_block(degp):
    deg = degp[0, :] + degp[1, :]
    return jax.lax.rsqrt(jnp.maximum(deg, 1.0))[:, None]


def _dense_a_body(degp_ref, feat_ref, hh_ref):
    hh_ref[...] = feat_ref[...] * _dinv_block(degp_ref[...])


def _dense_a(degp, feat_p):
    return pl.pallas_call(
        _dense_a_body,
        grid=(_GRID,),
        in_specs=[
            pl.BlockSpec((NCORES, _BLK), lambda i: (0, i)),
            pl.BlockSpec((_BLK, F), lambda i: (i, 0)),
        ],
        out_specs=pl.BlockSpec((_BLK, F), lambda i: (i, 0)),
        out_shape=jax.ShapeDtypeStruct((NP, F), jnp.float32),
    )(degp, feat_p)


def _dense_b_body(degp_ref, feat_ref, aggp_ref, feat2_ref, h2_ref, hh2_ref):
    dinv = _dinv_block(degp_ref[...])
    feat = feat_ref[...]
    agg = aggp_ref[0] + aggp_ref[1]
    feat2 = feat - agg * dinv
    feat2_ref[...] = feat2
    h2_ref[...] = 0.5 * feat + 0.4 * feat2
    hh2_ref[...] = feat2 * dinv


def _dense_b(degp, feat_p, aggp):
    return pl.pallas_call(
        _dense_b_body,
        grid=(_GRID,),
        in_specs=[
            pl.BlockSpec((NCORES, _BLK), lambda i: (0, i)),
            pl.BlockSpec((_BLK, F), lambda i: (i, 0)),
            pl.BlockSpec((NCORES, _BLK, F), lambda i: (0, i, 0)),
        ],
        out_specs=[pl.BlockSpec((_BLK, F), lambda i: (i, 0))] * 3,
        out_shape=[jax.ShapeDtypeStruct((NP, F), jnp.float32)] * 3,
    )(degp, feat_p, aggp)


def _dense_c_body(degp_ref, feat2_ref, h2_ref, aggp_ref, h_ref):
    dinv = _dinv_block(degp_ref[...])
    agg = aggp_ref[0] + aggp_ref[1]
    feat3 = feat2_ref[...] - agg * dinv
    h_ref[...] = h2_ref[...] + 0.1 * feat3


def _dense_c(degp, feat2, h2, aggp):
    return pl.pallas_call(
        _dense_c_body,
        grid=(_GRID,),
        in_specs=[
            pl.BlockSpec((NCORES, _BLK), lambda i: (0, i)),
            pl.BlockSpec((_BLK, F), lambda i: (i, 0)),
            pl.BlockSpec((_BLK, F), lambda i: (i, 0)),
            pl.BlockSpec((NCORES, _BLK, F), lambda i: (0, i, 0)),
        ],
        out_specs=pl.BlockSpec((_BLK, F), lambda i: (i, 0)),
        out_shape=jax.ShapeDtypeStruct((NP, F), jnp.float32),
    )(degp, feat2, h2, aggp)


def kernel(feat, edge_index, edge_mask):
    src = edge_index[0].astype(jnp.int32)
    dst = edge_index[1].astype(jnp.int32)
    pad = EPAD - E
    srcr = jnp.pad(src, (0, pad)).reshape(NWORK * CPW, CHUNK)
    dstr = jnp.pad(dst, (0, pad)).reshape(NWORK * CPW, CHUNK)
    maskr = jnp.pad(edge_mask, (0, pad)).reshape(NWORK * CPW, CHUNK)
    feat_p = jnp.pad(feat, ((0, NP - N), (0, 0)))

    degp = _deg_kernel(dstr, maskr).reshape(NCORES, NP)
    hh = _dense_a(degp, feat_p)
    aggp1 = _agg_kernel(hh, srcr, dstr, maskr)
    feat2, h2, hh2 = _dense_b(degp, feat_p, aggp1)
    aggp2 = _agg_kernel(hh2, srcr, dstr, maskr)
    h = _dense_c(degp, feat2, h2, aggp2)
    return h[:N]


# speedup vs baseline: 3.6806x; 1.1209x over previous
import functools

import jax
import jax.numpy as jnp
from jax import lax
from jax.experimental import pallas as pl
from jax.experimental.pallas import tpu as pltpu
from jax.experimental.pallas import tpu_sc as plsc

N = 10000
F = 128
E = 320000
NP = 10240
LANES = 16
NCORES = 2
NSUB = 16
NWORK = NCORES * NSUB
CHUNK = 128
CPW = 80
EPW = CPW * CHUNK
EPAD = NWORK * EPW
ROWS_PT = NP // NSUB

_mesh = plsc.VectorSubcoreMesh(core_axis_name="c", subcore_axis_name="s")


def _worker(c, s):
    return c * NSUB + s


@functools.partial(
    pl.kernel,
    out_type=jax.ShapeDtypeStruct((NCORES * NP,), jnp.float32),
    mesh=_mesh,
    scratch_types=[
        pltpu.VMEM((EPW,), jnp.int32),
        pltpu.VMEM((EPW,), jnp.float32),
        pltpu.VMEM((ROWS_PT,), jnp.float32),
        pltpu.VMEM_SHARED((NP,), jnp.float32),
    ],
)
def _deg_kernel(dstf_hbm, maskf_hbm, out_hbm, dst_v, mask_v, zero_v, deg_sh):
    c = lax.axis_index("c")
    s = lax.axis_index("s")
    w = _worker(c, s)

    @pl.loop(0, ROWS_PT // LANES)
    def _(i):
        zero_v[pl.ds(i * LANES, LANES)] = jnp.zeros((LANES,), jnp.float32)

    pltpu.sync_copy(zero_v, deg_sh.at[pl.ds(s * ROWS_PT, ROWS_PT)])

    base = w * EPW
    pltpu.sync_copy(dstf_hbm.at[pl.ds(base, EPW)], dst_v)
    pltpu.sync_copy(maskf_hbm.at[pl.ds(base, EPW)], mask_v)
    plsc.subcore_barrier()

    @pl.loop(0, EPW // 512)
    def _(t):
        off = t * 512
        pltpu.sync_copy(mask_v.at[pl.ds(off, 512)],
                        deg_sh.at[dst_v.at[pl.ds(off, 512)]], add=True)

    plsc.subcore_barrier()
    pltpu.sync_copy(deg_sh.at[pl.ds(s * ROWS_PT, ROWS_PT)],
                    out_hbm.at[pl.ds(c * NP + s * ROWS_PT, ROWS_PT)])


GRP = 128
NGRP = EPW // GRP


@functools.partial(
    pl.kernel,
    out_type=jax.ShapeDtypeStruct((NCORES, NP, F), jnp.float32),
    mesh=_mesh,
    scratch_types=[
        pltpu.VMEM((EPW,), jnp.int32),
        pltpu.VMEM((EPW,), jnp.int32),
        pltpu.VMEM((EPW,), jnp.float32),
        pltpu.VMEM((GRP, F), jnp.float32),
        pltpu.VMEM_SHARED((NP, F), jnp.float32),
    ],
)
def _agg_kernel(hh_hbm, srcf_hbm, dstf_hbm, maskf_hbm, out_hbm,
                src_v, dst_v, mask_v, rows, agg_sh):
    c = lax.axis_index("c")
    s = lax.axis_index("s")
    w = _worker(c, s)

    @pl.loop(0, GRP)
    def _(i):
        for v in range(F // LANES):
            rows[i, pl.ds(v * LANES, LANES)] = jnp.zeros((LANES,), jnp.float32)

    for k in range(ROWS_PT // GRP):
        pltpu.sync_copy(rows, agg_sh.at[pl.ds(s * ROWS_PT + k * GRP, GRP), :])
    if ROWS_PT % GRP:
        pltpu.sync_copy(rows.at[pl.ds(0, ROWS_PT % GRP), :],
                        agg_sh.at[pl.ds(s * ROWS_PT + (ROWS_PT // GRP) * GRP,
                                        ROWS_PT % GRP), :])

    base = w * EPW
    pltpu.sync_copy(srcf_hbm.at[pl.ds(base, EPW)], src_v)
    pltpu.sync_copy(dstf_hbm.at[pl.ds(base, EPW)], dst_v)
    pltpu.sync_copy(maskf_hbm.at[pl.ds(base, EPW)], mask_v)
    plsc.subcore_barrier()

    @pl.loop(0, NGRP)
    def _(t):
        off = t * GRP
        pltpu.sync_copy(hh_hbm.at[src_v.at[pl.ds(off, GRP)]], rows)

        @pl.loop(0, GRP // LANES)
        def _(g):
            mv = mask_v[pl.ds(off + g * LANES, LANES)]
            for l in range(LANES):
                m = mv[l]
                e = g * LANES + l
                for v in range(F // LANES):
                    sl = pl.ds(v * LANES, LANES)
                    rows[e, sl] = rows[e, sl] * m

        pltpu.sync_copy(rows, agg_sh.at[dst_v.at[pl.ds(off, GRP)]], add=True)

    plsc.subcore_barrier()
    pltpu.sync_copy(agg_sh.at[pl.ds(s * ROWS_PT, ROWS_PT), :],
                    out_hbm.at[c, pl.ds(s * ROWS_PT, ROWS_PT), :])


_BLK = 1024
_GRID = NP // _BLK


def _dinv_block(degp):
    deg = degp[0, :] + degp[1, :]
    return jax.lax.rsqrt(jnp.maximum(deg, 1.0))[:, None]


def _dense_a_body(degp_ref, feat_ref, hh_ref):
    hh_ref[...] = feat_ref[...] * _dinv_block(degp_ref[...])


def _dense_a(degp, feat_p):
    return pl.pallas_call(
        _dense_a_body,
        grid=(_GRID,),
        in_specs=[
            pl.BlockSpec((NCORES, _BLK), lambda i: (0, i)),
            pl.BlockSpec((_BLK, F), lambda i: (i, 0)),
        ],
        out_specs=pl.BlockSpec((_BLK, F), lambda i: (i, 0)),
        out_shape=jax.ShapeDtypeStruct((NP, F), jnp.float32),
    )(degp, feat_p)


def _dense_b_body(degp_ref, feat_ref, aggp_ref, feat2_ref, h2_ref, hh2_ref):
    dinv = _dinv_block(degp_ref[...])
    feat = feat_ref[...]
    agg = aggp_ref[0] + aggp_ref[1]
    feat2 = feat - agg * dinv
    feat2_ref[...] = feat2
    h2_ref[...] = 0.5 * feat + 0.4 * feat2
    hh2_ref[...] = feat2 * dinv


def _dense_b(degp, feat_p, aggp):
    return pl.pallas_call(
        _dense_b_body,
        grid=(_GRID,),
        in_specs=[
            pl.BlockSpec((NCORES, _BLK), lambda i: (0, i)),
            pl.BlockSpec((_BLK, F), lambda i: (i, 0)),
            pl.BlockSpec((NCORES, _BLK, F), lambda i: (0, i, 0)),
        ],
        out_specs=[pl.BlockSpec((_BLK, F), lambda i: (i, 0))] * 3,
        out_shape=[jax.ShapeDtypeStruct((NP, F), jnp.float32)] * 3,
    )(degp, feat_p, aggp)


def _dense_c_body(degp_ref, feat2_ref, h2_ref, aggp_ref, h_ref):
    dinv = _dinv_block(degp_ref[...])
    agg = aggp_ref[0] + aggp_ref[1]
    feat3 = feat2_ref[...] - agg * dinv
    h_ref[...] = h2_ref[...] + 0.1 * feat3


def _dense_c(degp, feat2, h2, aggp):
    return pl.pallas_call(
        _dense_c_body,
        grid=(_GRID,),
        in_specs=[
            pl.BlockSpec((NCORES, _BLK), lambda i: (0, i)),
            pl.BlockSpec((_BLK, F), lambda i: (i, 0)),
            pl.BlockSpec((_BLK, F), lambda i: (i, 0)),
            pl.BlockSpec((NCORES, _BLK, F), lambda i: (0, i, 0)),
        ],
        out_specs=pl.BlockSpec((_BLK, F), lambda i: (i, 0)),
        out_shape=jax.ShapeDtypeStruct((NP, F), jnp.float32),
    )(degp, feat2, h2, aggp)


def kernel(feat, edge_index, edge_mask):
    src = edge_index[0].astype(jnp.int32)
    dst = edge_index[1].astype(jnp.int32)
    pad = EPAD - E
    srcf = jnp.pad(src, (0, pad))
    dstf = jnp.pad(dst, (0, pad))
    maskf = jnp.pad(edge_mask, (0, pad))
    feat_p = jnp.pad(feat, ((0, NP - N), (0, 0)))

    degp = _deg_kernel(dstf, maskf).reshape(NCORES, NP)
    hh = _dense_a(degp, feat_p)
    aggp1 = _agg_kernel(hh, srcf, dstf, maskf)
    feat2, h2, hh2 = _dense_b(degp, feat_p, aggp1)
    aggp2 = _agg_kernel(hh2, srcf, dstf, maskf)
    h = _dense_c(degp, feat2, h2, aggp2)
    return h[:N]
